# baseline (speedup 1.0000x reference)
import jax
import jax.numpy as jnp
from jax import lax
from jax.experimental import pallas as pl
from jax.experimental.pallas import tpu as pltpu
from jax.experimental.pallas import tpu_sc as plsc

_N = 10000
_NPAD = 10240
_D = 128
_E = 320000
_NC = 2
_NS = 16
_NW = _NC * _NS
_CH = 128
_G = 80
_G0 = 96
_G1 = 2 * _G - _G0
_EPAD = _NW * _G * _CH
_RPT = _NPAD // _NS
_ZR = 128
_NDUMP = _RPT // _ZR
_BLK = 256


def _sc_agg_body(with_counts, *refs):
    if with_counts:
        (table, src, dst, agg_out, cnt_out,
         srcw0, srcw1, dstw0, dstw1, buf0, buf1, cnt_v, agg_sh,
         sem_g0, sem_g1, sem_s0, sem_s1, sem_i0, sem_i1) = refs
    else:
        (table, src, dst, agg_out,
         srcw0, srcw1, dstw0, dstw1, buf0, buf1, agg_sh,
         sem_g0, sem_g1, sem_s0, sem_s1, sem_i0, sem_i1) = refs
    c = lax.axis_index("c")
    s = lax.axis_index("s")
    zeros16 = jnp.zeros((16,), jnp.float32)
    ones16 = jnp.ones((16,), jnp.float32)

    def _zb(i, carry):
        r = i // (_D // 16)
        col = (i % (_D // 16)) * 16
        buf0[r, pl.ds(col, 16)] = zeros16
        return carry
    lax.fori_loop(0, _ZR * _D // 16, _zb, 0)
    base = s * _RPT
    for r in range(_NDUMP):
        pltpu.sync_copy(buf0, agg_sh.at[pl.ds(base + r * _ZR, _ZR)])
    if with_counts:
        def _zc(i, carry):
            cnt_v[pl.ds(i * 16, 16)] = zeros16
            return carry
        lax.fori_loop(0, _NPAD // 16, _zc, 0)
    plsc.subcore_barrier()

    def _fire_idx(j, g_lim, ebase, edges, w, sem):
        jc = jnp.minimum(j, g_lim - 1)
        pltpu.async_copy(edges.at[pl.ds(ebase + jc * _CH, _CH)], w, sem)

    def _drain_idx(w, sem):
        pltpu.make_async_copy(src.at[pl.ds(0, _CH)], w, sem).wait()

    def _fire_gather(w, buf, sem):
        pltpu.async_copy(table.at[w], buf, sem)

    def _drain_gather(buf, sem):
        pltpu.make_async_copy(table.at[pl.ds(0, _CH)], buf, sem).wait()

    def _fire_scatter(w, buf, sem):
        pltpu.async_copy(buf, agg_sh.at[w], sem, add=True)

    def _drain_scatter(buf, sem):
        pltpu.make_async_copy(buf, agg_sh.at[pl.ds(0, _CH)], sem).wait()

    def _counts(w):
        if with_counts:
            for k in range(_CH // 16):
                idx = w[pl.ds(k * 16, 16)]
                plsc.addupdate_scatter(cnt_v, [idx], ones16)

    def _run_pipelined():
        ebase = s * _G0 * _CH
        _fire_idx(0, _G0, ebase, src, srcw0, sem_i0)
        _fire_idx(0, _G0, ebase, dst, dstw0, sem_i0)
        _fire_idx(1, _G0, ebase, src, srcw1, sem_i1)
        _fire_idx(1, _G0, ebase, dst, dstw1, sem_i1)
        _drain_idx(srcw0, sem_i0)
        _drain_idx(dstw0, sem_i0)
        _fire_gather(srcw0, buf0, sem_g0)
        _drain_idx(srcw1, sem_i1)
        _drain_idx(dstw1, sem_i1)
        _fire_gather(srcw1, buf1, sem_g1)

        def _pipe(it, carry):
            j0 = it * 2
            _drain_gather(buf0, sem_g0)
            _fire_idx(j0 + 2, _G0, ebase, src, srcw0, sem_i0)
            _fire_scatter(dstw0, buf0, sem_s0)
            _counts(dstw0)
            _drain_gather(buf1, sem_g1)
            _fire_idx(j0 + 3, _G0, ebase, src, srcw1, sem_i1)
            _drain_scatter(buf0, sem_s0)
            _fire_idx(j0 + 2, _G0, ebase, dst, dstw0, sem_i0)
            _fire_scatter(dstw1, buf1, sem_s1)
            _counts(dstw1)
            _drain_idx(srcw0, sem_i0)
            _drain_idx(dstw0, sem_i0)
            _fire_gather(srcw0, buf0, sem_g0)
            _drain_scatter(buf1, sem_s1)
            _fire_idx(j0 + 3, _G0, ebase, dst, dstw1, sem_i1)
            _drain_idx(srcw1, sem_i1)
            _drain_idx(dstw1, sem_i1)
            _fire_gather(srcw1, buf1, sem_g1)
            return carry
        lax.fori_loop(0, _G0 // 2, _pipe, 0)
        _drain_gather(buf0, sem_g0)
        _drain_gather(buf1, sem_g1)

    def _run_sync():
        ebase = (_NS * _G0 + s * _G1) * _CH

        def _step(j, carry):
            b = ebase + j * _CH
            pltpu.sync_copy(src.at[pl.ds(b, _CH)], srcw0)
            pltpu.sync_copy(dst.at[pl.ds(b, _CH)], dstw0)
            pltpu.async_copy(table.at[srcw0], buf0, sem_g0).wait()
            pltpu.sync_copy(buf0, agg_sh.at[dstw0], add=True)
            _counts(dstw0)
            return carry
        lax.fori_loop(0, _G1, _step, 0)

    pl.when(c == 0)(_run_pipelined)
    pl.when(c != 0)(_run_sync)
    plsc.subcore_barrier()

    for r in range(_NDUMP):
        b = buf0 if r % 2 == 0 else buf1
        pltpu.sync_copy(agg_sh.at[pl.ds(base + r * _ZR, _ZR)], b)
        pltpu.sync_copy(b, agg_out.at[c, pl.ds(base + r * _ZR, _ZR)])
    if with_counts:
        pltpu.sync_copy(cnt_v, cnt_out.at[c, s])


def _make_sc_agg(with_counts):
    mesh = plsc.VectorSubcoreMesh(core_axis_name="c", subcore_axis_name="s",
                                  num_cores=_NC, num_subcores=_NS)
    out_type = [jax.ShapeDtypeStruct((_NC, _NPAD, _D), jnp.float32)]
    if with_counts:
        out_type.append(jax.ShapeDtypeStruct((_NC, _NS, _NPAD), jnp.float32))
    scratch = [
        pltpu.VMEM((_CH,), jnp.int32),
        pltpu.VMEM((_CH,), jnp.int32),
        pltpu.VMEM((_CH,), jnp.int32),
        pltpu.VMEM((_CH,), jnp.int32),
        pltpu.VMEM((_CH, _D), jnp.float32),
        pltpu.VMEM((_CH, _D), jnp.float32),
    ]
    if with_counts:
        scratch.append(pltpu.VMEM((_NPAD,), jnp.float32))
    scratch.append(pltpu.VMEM_SHARED((_NPAD, _D), jnp.float32))
    scratch.extend([pltpu.SemaphoreType.DMA] * 6)

    def body(*refs):
        _sc_agg_body(with_counts, *refs)
    return pl.kernel(body, out_type=tuple(out_type), mesh=mesh,
                     compiler_params=pltpu.CompilerParams(needs_layout_passes=False),
                     scratch_types=tuple(scratch))


_SC_CACHE = {}


def _sc_agg(with_counts, *args):
    if with_counts not in _SC_CACHE:
        _SC_CACHE[with_counts] = _make_sc_agg(with_counts)
    return _SC_CACHE[with_counts](*args)


def _tc_layer1_body(agg_ref, cnt_ref, x_ref, wl_ref, wr_ref, b_ref, out_ref):
    agg = agg_ref[0] + agg_ref[1]
    cnt = jnp.sum(cnt_ref[...].reshape(_NC * _NS, _BLK), axis=0)
    inv = 1.0 / jnp.maximum(cnt, 1.0)
    mean = agg * inv[:, None]
    h = lax.dot_general(mean, wl_ref[...], (((1,), (1,)), ((), ())),
                        preferred_element_type=jnp.float32)
    h = h + lax.dot_general(x_ref[...], wr_ref[...], (((1,), (1,)), ((), ())),
                            preferred_element_type=jnp.float32)
    h = h + b_ref[...]
    out_ref[...] = jnp.maximum(h, 0.0)


def _tc_layer2_body(agg_ref, cnt_ref, x_ref, wl_ref, wr_ref, b_ref,
                    wo_ref, bo_ref, h_ref, logit_ref):
    agg = agg_ref[0] + agg_ref[1]
    cnt = jnp.sum(cnt_ref[...].reshape(_NC * _NS, _BLK), axis=0)
    inv = 1.0 / jnp.maximum(cnt, 1.0)
    mean = agg * inv[:, None]
    h = lax.dot_general(mean, wl_ref[...], (((1,), (1,)), ((), ())),
                        preferred_element_type=jnp.float32)
    h = h + lax.dot_general(x_ref[...], wr_ref[...], (((1,), (1,)), ((), ())),
                            preferred_element_type=jnp.float32)
    h = h + b_ref[...]
    h = jnp.maximum(h, 0.0)
    h_ref[...] = h
    logit_ref[...] = lax.dot_general(h, wo_ref[...], (((1,), (1,)), ((), ())),
                                     preferred_element_type=jnp.float32) + bo_ref[...]


_agg_spec = pl.BlockSpec((_NC, _BLK, _D), lambda i: (0, i, 0))
_cnt_spec = pl.BlockSpec((_NC, _NS, _BLK), lambda i: (0, 0, i))
_row_spec = pl.BlockSpec((_BLK, _D), lambda i: (i, 0))
_w_spec = pl.BlockSpec((_D, _D), lambda i: (0, 0))
_b_spec = pl.BlockSpec((1, _D), lambda i: (0, 0))


def _tc_layer1(agg, cnt, x, wl, wr, b):
    return pl.pallas_call(
        _tc_layer1_body,
        grid=(_NPAD // _BLK,),
        in_specs=[_agg_spec, _cnt_spec, _row_spec, _w_spec, _w_spec, _b_spec],
        out_specs=_row_spec,
        out_shape=jax.ShapeDtypeStruct((_NPAD, _D), jnp.float32),
    )(agg, cnt, x, wl, wr, b)


def _tc_layer2(agg, cnt, h1, wl, wr, b, wo, bo):
    return pl.pallas_call(
        _tc_layer2_body,
        grid=(_NPAD // _BLK,),
        in_specs=[_agg_spec, _cnt_spec, _row_spec, _w_spec, _w_spec, _b_spec,
                  _w_spec, _b_spec],
        out_specs=(_row_spec, _row_spec),
        out_shape=(jax.ShapeDtypeStruct((_NPAD, _D), jnp.float32),
                   jax.ShapeDtypeStruct((_NPAD, _D), jnp.float32)),
    )(agg, cnt, h1, wl, wr, b, wo, bo)


def kernel(x, edge_index, W1l, b1l, W1r, W2l, b2l, W2r, Wo, bo):
    src = edge_index[0]
    dst = edge_index[1]
    pad = jnp.full((_EPAD - _E,), _N, dtype=jnp.int32)
    srcp = jnp.concatenate([src, pad])
    dstp = jnp.concatenate([dst, pad])
    xp = jnp.zeros((_NPAD, _D), jnp.float32).at[:_N].set(x)
    agg1, cnt = _sc_agg(True, xp, srcp, dstp)
    h1 = _tc_layer1(agg1, cnt, xp, W1l, W1r, b1l.reshape(1, _D))
    (agg2,) = _sc_agg(False, h1, srcp, dstp)
    wo_pad = jnp.zeros((_D, _D), jnp.float32).at[:Wo.shape[0]].set(Wo)
    bo_pad = jnp.zeros((1, _D), jnp.float32).at[0, :bo.shape[0]].set(bo)
    h2, logits_pad = _tc_layer2(agg2, cnt, h1, W2l, W2r, b2l.reshape(1, _D),
                                wo_pad, bo_pad)
    return (logits_pad[:_N, :Wo.shape[0]], h2[:_N])


# speedup vs baseline: 1.3622x; 1.3622x over previous
import jax
import jax.numpy as jnp
from jax import lax
from jax.experimental import pallas as pl
from jax.experimental.pallas import tpu as pltpu
from jax.experimental.pallas import tpu_sc as plsc

_N = 10000
_NPAD = 10240
_D = 128
_E = 320000
_NC = 2
_NS = 16
_NW = _NC * _NS
_CH = 128
_G = 80
_EPAD = _NW * _G * _CH
_RPT = _NPAD // _NS
_ZR = 128
_NDUMP = _RPT // _ZR
_BLK = 256


def _sc_agg_body(with_counts, *refs):
    if with_counts:
        (table, src, dst, agg_out, cnt_out,
         srcw0, srcw1, dstw0, dstw1, bfa, bfb, fbuf, cnt_v, agg_sh,
         sem_g0, sem_g1, sem_s0, sem_i0, sem_i1) = refs
    else:
        (table, src, dst, agg_out,
         srcw0, srcw1, dstw0, dstw1, bfa, bfb, fbuf, agg_sh,
         sem_g0, sem_g1, sem_s0, sem_i0, sem_i1) = refs
    c = lax.axis_index("c")
    s = lax.axis_index("s")
    wid = s * _NC + c
    ebase = wid * _G * _CH
    zeros16 = jnp.zeros((16,), jnp.float32)
    ones16 = jnp.ones((16,), jnp.float32)

    def _zb(i, carry):
        r = i // (_D // 16)
        col = (i % (_D // 16)) * 16
        fbuf[r, pl.ds(col, 16)] = zeros16
        return carry
    lax.fori_loop(0, _ZR * _D // 16, _zb, 0)
    base = s * _RPT
    for r in range(_NDUMP):
        pltpu.sync_copy(fbuf, agg_sh.at[pl.ds(base + r * _ZR, _ZR)])
    if with_counts:
        def _zc(i, carry):
            cnt_v[pl.ds(i * 16, 16)] = zeros16
            return carry
        lax.fori_loop(0, _NPAD // 16, _zc, 0)
    plsc.subcore_barrier()

    def _fire_idx(j, edges, w, sem):
        jc = jnp.minimum(j, _G - 1)
        pltpu.async_copy(edges.at[pl.ds(ebase + jc * _CH, _CH)], w, sem)

    def _drain_idx(w, sem):
        pltpu.make_async_copy(src.at[pl.ds(0, _CH)], w, sem).wait()

    def _fire_gather(w, bf, sem):
        pltpu.async_copy(table.at[w], bf, sem)

    def _drain_gather(bf, sem):
        pltpu.make_async_copy(table.at[pl.ds(0, _CH)], bf, sem).wait()

    def _fire_scatter(w, sem):
        pltpu.async_copy(fbuf, agg_sh.at[w], sem, add=True)

    def _drain_scatter(sem):
        pltpu.make_async_copy(fbuf, agg_sh.at[pl.ds(0, _CH)], sem).wait()

    def _convert(bf):
        def _cv(r, carry):
            for k in range(_D // 32):
                v = bf[r, pl.ds(k * 16, 16)]
                lo = plsc.bitcast(v << 16, jnp.float32)
                hi = plsc.bitcast(v & jnp.int32(-65536), jnp.float32)
                fbuf[r, pl.ds(k * 32, 16)] = lo
                fbuf[r, pl.ds(k * 32 + 16, 16)] = hi
            return carry
        lax.fori_loop(0, _CH, _cv, 0)

    def _counts(w):
        if with_counts:
            for k in range(_CH // 16):
                idx = w[pl.ds(k * 16, 16)]
                plsc.addupdate_scatter(cnt_v, [idx], ones16)

    _fire_idx(0, src, srcw0, sem_i0)
    _fire_idx(0, dst, dstw0, sem_i0)
    _fire_idx(1, src, srcw1, sem_i1)
    _fire_idx(1, dst, dstw1, sem_i1)
    _drain_idx(srcw0, sem_i0)
    _drain_idx(dstw0, sem_i0)
    _fire_gather(srcw0, bfa, sem_g0)
    _drain_idx(srcw1, sem_i1)
    _drain_idx(dstw1, sem_i1)
    _fire_gather(srcw1, bfb, sem_g1)

    def _pipe(it, carry):
        j0 = it * 2
        _drain_gather(bfa, sem_g0)
        _fire_idx(j0 + 2, src, srcw0, sem_i0)
        _convert(bfa)
        _fire_scatter(dstw0, sem_s0)
        _counts(dstw0)
        _drain_scatter(sem_s0)
        _fire_idx(j0 + 2, dst, dstw0, sem_i0)
        _drain_gather(bfb, sem_g1)
        _drain_idx(srcw0, sem_i0)
        _drain_idx(dstw0, sem_i0)
        _fire_gather(srcw0, bfa, sem_g0)
        _fire_idx(j0 + 3, src, srcw1, sem_i1)
        _convert(bfb)
        _fire_scatter(dstw1, sem_s0)
        _counts(dstw1)
        _drain_scatter(sem_s0)
        _fire_idx(j0 + 3, dst, dstw1, sem_i1)
        _drain_idx(srcw1, sem_i1)
        _drain_idx(dstw1, sem_i1)
        _fire_gather(srcw1, bfb, sem_g1)
        return carry
    lax.fori_loop(0, _G // 2, _pipe, 0)
    _drain_gather(bfa, sem_g0)
    _drain_gather(bfb, sem_g1)
    plsc.subcore_barrier()

    for r in range(_NDUMP):
        pltpu.sync_copy(agg_sh.at[pl.ds(base + r * _ZR, _ZR)], fbuf)
        pltpu.sync_copy(fbuf, agg_out.at[c, pl.ds(base + r * _ZR, _ZR)])
    if with_counts:
        pltpu.sync_copy(cnt_v, cnt_out.at[c, s])


def _make_sc_agg(with_counts):
    mesh = plsc.VectorSubcoreMesh(core_axis_name="c", subcore_axis_name="s",
                                  num_cores=_NC, num_subcores=_NS)
    out_type = [jax.ShapeDtypeStruct((_NC, _NPAD, _D), jnp.float32)]
    if with_counts:
        out_type.append(jax.ShapeDtypeStruct((_NC, _NS, _NPAD), jnp.float32))
    scratch = [
        pltpu.VMEM((_CH,), jnp.int32),
        pltpu.VMEM((_CH,), jnp.int32),
        pltpu.VMEM((_CH,), jnp.int32),
        pltpu.VMEM((_CH,), jnp.int32),
        pltpu.VMEM((_CH, _D // 2), jnp.int32),
        pltpu.VMEM((_CH, _D // 2), jnp.int32),
        pltpu.VMEM((_CH, _D), jnp.float32),
    ]
    if with_counts:
        scratch.append(pltpu.VMEM((_NPAD,), jnp.float32))
    scratch.append(pltpu.VMEM_SHARED((_NPAD, _D), jnp.float32))
    scratch.extend([pltpu.SemaphoreType.DMA] * 5)

    def body(*refs):
        _sc_agg_body(with_counts, *refs)
    return pl.kernel(body, out_type=tuple(out_type), mesh=mesh,
                     compiler_params=pltpu.CompilerParams(
                         needs_layout_passes=False, use_tc_tiling_on_sc=False),
                     scratch_types=tuple(scratch))


_SC_CACHE = {}


def _sc_agg(with_counts, *args):
    if with_counts not in _SC_CACHE:
        _SC_CACHE[with_counts] = _make_sc_agg(with_counts)
    return _SC_CACHE[with_counts](*args)


def _tc_layer1_body(agg_ref, cnt_ref, x_ref, wl_ref, wr_ref, b_ref,
                    out_ref, outb_ref):
    agg = agg_ref[0] + agg_ref[1]
    cnt = jnp.sum(cnt_ref[...].reshape(_NC * _NS, _BLK), axis=0)
    inv = 1.0 / jnp.maximum(cnt, 1.0)
    mean = agg * inv[:, None]
    h = lax.dot_general(mean, wl_ref[...], (((1,), (1,)), ((), ())),
                        preferred_element_type=jnp.float32)
    h = h + lax.dot_general(x_ref[...], wr_ref[...], (((1,), (1,)), ((), ())),
                            preferred_element_type=jnp.float32)
    h = h + b_ref[...]
    h = jnp.maximum(h, 0.0)
    out_ref[...] = h
    outb_ref[...] = h.astype(jnp.bfloat16)


def _tc_layer2_body(agg_ref, cnt_ref, x_ref, wl_ref, wr_ref, b_ref,
                    wo_ref, bo_ref, h_ref, logit_ref):
    agg = agg_ref[0] + agg_ref[1]
    cnt = jnp.sum(cnt_ref[...].reshape(_NC * _NS, _BLK), axis=0)
    inv = 1.0 / jnp.maximum(cnt, 1.0)
    mean = agg * inv[:, None]
    h = lax.dot_general(mean, wl_ref[...], (((1,), (1,)), ((), ())),
                        preferred_element_type=jnp.float32)
    h = h + lax.dot_general(x_ref[...], wr_ref[...], (((1,), (1,)), ((), ())),
                            preferred_element_type=jnp.float32)
    h = h + b_ref[...]
    h = jnp.maximum(h, 0.0)
    h_ref[...] = h
    logit_ref[...] = lax.dot_general(h, wo_ref[...], (((1,), (1,)), ((), ())),
                                     preferred_element_type=jnp.float32) + bo_ref[...]


_agg_spec = pl.BlockSpec((_NC, _BLK, _D), lambda i: (0, i, 0))
_cnt_spec = pl.BlockSpec((_NC, _NS, _BLK), lambda i: (0, 0, i))
_row_spec = pl.BlockSpec((_BLK, _D), lambda i: (i, 0))
_w_spec = pl.BlockSpec((_D, _D), lambda i: (0, 0))
_b_spec = pl.BlockSpec((1, _D), lambda i: (0, 0))


def _tc_layer1(agg, cnt, x, wl, wr, b):
    return pl.pallas_call(
        _tc_layer1_body,
        grid=(_NPAD // _BLK,),
        in_specs=[_agg_spec, _cnt_spec, _row_spec, _w_spec, _w_spec, _b_spec],
        out_specs=(_row_spec, _row_spec),
        out_shape=(jax.ShapeDtypeStruct((_NPAD, _D), jnp.float32),
                   jax.ShapeDtypeStruct((_NPAD, _D), jnp.bfloat16)),
    )(agg, cnt, x, wl, wr, b)


def _tc_layer2(agg, cnt, h1, wl, wr, b, wo, bo):
    return pl.pallas_call(
        _tc_layer2_body,
        grid=(_NPAD // _BLK,),
        in_specs=[_agg_spec, _cnt_spec, _row_spec, _w_spec, _w_spec, _b_spec,
                  _w_spec, _b_spec],
        out_specs=(_row_spec, _row_spec),
        out_shape=(jax.ShapeDtypeStruct((_NPAD, _D), jnp.float32),
                   jax.ShapeDtypeStruct((_NPAD, _D), jnp.float32)),
    )(agg, cnt, h1, wl, wr, b, wo, bo)


def kernel(x, edge_index, W1l, b1l, W1r, W2l, b2l, W2r, Wo, bo):
    src = edge_index[0]
    dst = edge_index[1]
    pad = jnp.full((_EPAD - _E,), _N, dtype=jnp.int32)
    srcp = jnp.concatenate([src, pad])
    dstp = jnp.concatenate([dst, pad])
    xp = jnp.zeros((_NPAD, _D), jnp.float32).at[:_N].set(x)
    xb = lax.bitcast_convert_type(
        xp.astype(jnp.bfloat16).reshape(_NPAD, _D // 2, 2), jnp.int32)
    perm = (jnp.arange(_D) // 32) * 32 + jnp.where(
        (jnp.arange(_D) % 32) < 16, (jnp.arange(_D) % 32) * 2,
        ((jnp.arange(_D) % 32) - 16) * 2 + 1)
    agg1, cnt = _sc_agg(True, xb, srcp, dstp)
    h1, h1b = _tc_layer1(agg1, cnt, xp, W1l[:, perm], W1r, b1l.reshape(1, _D))
    h1b32 = lax.bitcast_convert_type(h1b.reshape(_NPAD, _D // 2, 2), jnp.int32)
    (agg2,) = _sc_agg(False, h1b32, srcp, dstp)
    wo_pad = jnp.zeros((_D, _D), jnp.float32).at[:Wo.shape[0]].set(Wo)
    bo_pad = jnp.zeros((1, _D), jnp.float32).at[0, :bo.shape[0]].set(bo)
    h2, logits_pad = _tc_layer2(agg2, cnt, h1, W2l[:, perm], W2r,
                                b2l.reshape(1, _D), wo_pad, bo_pad)
    return (logits_pad[:_N, :Wo.shape[0]], h2[:_N])
